# baseline (speedup 1.0000x reference)
import jax
import jax.numpy as jnp
from jax.experimental import pallas as pl
from jax.experimental.pallas import tpu as pltpu

_NUM_LEVELS = 4
_D = 64
_K = 1024
_BETA = 0.25
_NB = 2048
_CH = 256


def _rvq_body(z_ref, cbs_ref, zq_ref, idx_ref, sse_ref):
    n_chunks = _NB // _CH
    iota_k = jax.lax.broadcasted_iota(jnp.int32, (_CH, _K), 1)
    rs = [z_ref[c * _CH:(c + 1) * _CH, :] for c in range(n_chunks)]
    zqs = [jnp.zeros((_CH, _D), jnp.float32) for _ in range(n_chunks)]
    sses = [jnp.zeros((1, 1), jnp.float32) for _ in range(n_chunks)]
    idx_cols = [[] for _ in range(n_chunks)]
    for lvl in range(_NUM_LEVELS):
        cb = cbs_ref[lvl]
        c2 = jnp.sum(cb * cb, axis=1)
        cb_bf = cb.astype(jnp.bfloat16)
        r1 = cb - cb_bf.astype(jnp.float32)
        cb_mid = r1.astype(jnp.bfloat16)
        cb_lo = (r1 - cb_mid.astype(jnp.float32)).astype(jnp.bfloat16)
        for c in range(n_chunks):
            r = rs[c]
            s = jax.lax.dot_general(
                r.astype(jnp.bfloat16), cb_bf,
                (((1,), (1,)), ((), ())),
                preferred_element_type=jnp.float32)
            d2 = (jnp.sum(r * r, axis=1, keepdims=True) - 2.0 * s + c2[None, :])
            idx = jnp.argmin(d2, axis=1)
            idx_cols[c].append(idx)
            onehot = (iota_k == idx[:, None]).astype(jnp.bfloat16)

            def pick(part, onehot=onehot):
                return jax.lax.dot_general(
                    onehot, part, (((1,), (0,)), ((), ())),
                    preferred_element_type=jnp.float32)

            q = (pick(cb_bf) + pick(cb_mid)) + pick(cb_lo)
            diff = q - r
            sses[c] = sses[c] + jnp.sum(diff * diff, keepdims=True)
            q_st = r + diff
            zqs[c] = zqs[c] + q_st
            rs[c] = r - q_st
    for c in range(n_chunks):
        zq_ref[c * _CH:(c + 1) * _CH, :] = zqs[c]
        idx_ref[c * _CH:(c + 1) * _CH, :] = jnp.stack(idx_cols[c], axis=1)
    sse = sum(sses[1:], sses[0])
    sse_ref[...] = jnp.broadcast_to(sse.reshape(1, 1, 1), (1, 1, 128))


def kernel(z, codebook_0, codebook_1, codebook_2, codebook_3):
    orig_shape = z.shape
    n = orig_shape[0] * orig_shape[1]
    z_flat = z.reshape(n, _D)
    cbs = jnp.stack([codebook_0, codebook_1, codebook_2, codebook_3], axis=0)

    nblk = n // _NB
    zq_flat, idx_flat, sse_parts = pl.pallas_call(
        _rvq_body,
        grid=(nblk,),
        in_specs=[
            pl.BlockSpec((_NB, _D), lambda i: (i, 0)),
            pl.BlockSpec((_NUM_LEVELS, _K, _D), lambda i: (0, 0, 0)),
        ],
        out_specs=[
            pl.BlockSpec((_NB, _D), lambda i: (i, 0)),
            pl.BlockSpec((_NB, _NUM_LEVELS), lambda i: (i, 0)),
            pl.BlockSpec((1, 1, 128), lambda i: (i, 0, 0)),
        ],
        out_shape=[
            jax.ShapeDtypeStruct((n, _D), jnp.float32),
            jax.ShapeDtypeStruct((n, _NUM_LEVELS), jnp.int32),
            jax.ShapeDtypeStruct((nblk, 1, 128), jnp.float32),
        ],
        compiler_params=pltpu.CompilerParams(
            dimension_semantics=("parallel",)),
    )(z_flat, cbs)

    codebook_loss = (jnp.sum(sse_parts[:, 0, 0]) / jnp.float32(n * _D)).astype(jnp.float32)
    commitment_loss = jnp.float32(_BETA) * codebook_loss
    z_q = zq_flat.reshape(orig_shape)
    indices_out = idx_flat.reshape(orig_shape[:-1] + (_NUM_LEVELS,))
    return z_q, indices_out, commitment_loss, codebook_loss


# speedup vs baseline: 1.0167x; 1.0167x over previous
import jax
import jax.numpy as jnp
from jax.experimental import pallas as pl
from jax.experimental.pallas import tpu as pltpu

_NUM_LEVELS = 4
_D = 64
_K = 1024
_BETA = 0.25
_NB = 1024
_CH = 256


def _rvq_body(z_ref, cbs_ref, zq_ref, idx_ref, sse_ref):
    n_chunks = _NB // _CH
    iota_k = jax.lax.broadcasted_iota(jnp.int32, (_CH, _K), 1)
    rs = [z_ref[c * _CH:(c + 1) * _CH, :] for c in range(n_chunks)]
    zqs = [jnp.zeros((_CH, _D), jnp.float32) for _ in range(n_chunks)]
    sses = [jnp.zeros((1, 1), jnp.float32) for _ in range(n_chunks)]
    idx_cols = [[] for _ in range(n_chunks)]
    for lvl in range(_NUM_LEVELS):
        cb = cbs_ref[lvl]
        c2 = jnp.sum(cb * cb, axis=1)
        cb_bf = cb.astype(jnp.bfloat16)
        r1 = cb - cb_bf.astype(jnp.float32)
        cb_mid = r1.astype(jnp.bfloat16)
        cb_lo = (r1 - cb_mid.astype(jnp.float32)).astype(jnp.bfloat16)
        for c in range(n_chunks):
            r = rs[c]
            s = jax.lax.dot_general(
                r.astype(jnp.bfloat16), cb_bf,
                (((1,), (1,)), ((), ())),
                preferred_element_type=jnp.float32)
            d2 = (jnp.sum(r * r, axis=1, keepdims=True) - 2.0 * s + c2[None, :])
            m = jnp.min(d2, axis=1, keepdims=True)
            t = jnp.where(d2 == m, iota_k, _K)
            idx = jnp.min(t, axis=1)
            idx_cols[c].append(idx)
            onehot = (iota_k == idx[:, None]).astype(jnp.bfloat16)

            def pick(part, onehot=onehot):
                return jax.lax.dot_general(
                    onehot, part, (((1,), (0,)), ((), ())),
                    preferred_element_type=jnp.float32)

            q = (pick(cb_bf) + pick(cb_mid)) + pick(cb_lo)
            diff = q - r
            sses[c] = sses[c] + jnp.sum(diff * diff, keepdims=True)
            q_st = r + diff
            zqs[c] = zqs[c] + q_st
            rs[c] = r - q_st
    for c in range(n_chunks):
        zq_ref[c * _CH:(c + 1) * _CH, :] = zqs[c]
        idx_ref[c * _CH:(c + 1) * _CH, :] = jnp.stack(idx_cols[c], axis=1)
    sse = sum(sses[1:], sses[0])
    sse_ref[...] = jnp.broadcast_to(sse.reshape(1, 1, 1), (1, 1, 128))


def kernel(z, codebook_0, codebook_1, codebook_2, codebook_3):
    orig_shape = z.shape
    n = orig_shape[0] * orig_shape[1]
    z_flat = z.reshape(n, _D)
    cbs = jnp.stack([codebook_0, codebook_1, codebook_2, codebook_3], axis=0)

    nblk = n // _NB
    zq_flat, idx_flat, sse_parts = pl.pallas_call(
        _rvq_body,
        grid=(nblk,),
        in_specs=[
            pl.BlockSpec((_NB, _D), lambda i: (i, 0)),
            pl.BlockSpec((_NUM_LEVELS, _K, _D), lambda i: (0, 0, 0)),
        ],
        out_specs=[
            pl.BlockSpec((_NB, _D), lambda i: (i, 0)),
            pl.BlockSpec((_NB, _NUM_LEVELS), lambda i: (i, 0)),
            pl.BlockSpec((1, 1, 128), lambda i: (i, 0, 0)),
        ],
        out_shape=[
            jax.ShapeDtypeStruct((n, _D), jnp.float32),
            jax.ShapeDtypeStruct((n, _NUM_LEVELS), jnp.int32),
            jax.ShapeDtypeStruct((nblk, 1, 128), jnp.float32),
        ],
        compiler_params=pltpu.CompilerParams(
            dimension_semantics=("parallel",)),
    )(z_flat, cbs)

    codebook_loss = (jnp.sum(sse_parts[:, 0, 0]) / jnp.float32(n * _D)).astype(jnp.float32)
    commitment_loss = jnp.float32(_BETA) * codebook_loss
    z_q = zq_flat.reshape(orig_shape)
    indices_out = idx_flat.reshape(orig_shape[:-1] + (_NUM_LEVELS,))
    return z_q, indices_out, commitment_loss, codebook_loss


# speedup vs baseline: 1.1171x; 1.0988x over previous
import jax
import jax.numpy as jnp
from jax.experimental import pallas as pl
from jax.experimental.pallas import tpu as pltpu

_NUM_LEVELS = 4
_D = 64
_K = 1024
_BETA = 0.25
_NB = 1024
_CH = 256


def _rvq_body(z_ref, cbs_ref, zq_ref, idx_ref, sse_ref):
    n_chunks = _NB // _CH
    iota_k = jax.lax.broadcasted_iota(jnp.int32, (_CH, _K), 1)
    iota_kf = iota_k.astype(jnp.float32)
    rs = [z_ref[c * _CH:(c + 1) * _CH, :] for c in range(n_chunks)]
    zqs = [jnp.zeros((_CH, _D), jnp.float32) for _ in range(n_chunks)]
    sses = [jnp.zeros((1, 1), jnp.float32) for _ in range(n_chunks)]
    idx_cols = [[] for _ in range(n_chunks)]
    for lvl in range(_NUM_LEVELS):
        cb = cbs_ref[lvl]
        c2 = jnp.sum(cb * cb, axis=1)
        cb_bf = cb.astype(jnp.bfloat16)
        r1 = cb - cb_bf.astype(jnp.float32)
        cb_mid = r1.astype(jnp.bfloat16)
        cb_lo = (r1 - cb_mid.astype(jnp.float32)).astype(jnp.bfloat16)
        for c in range(n_chunks):
            r = rs[c]
            s = jax.lax.dot_general(
                r.astype(jnp.bfloat16), cb_bf,
                (((1,), (1,)), ((), ())),
                preferred_element_type=jnp.float32)
            d2 = (jnp.sum(r * r, axis=1, keepdims=True) - 2.0 * s + c2[None, :])
            m = jnp.min(d2, axis=1, keepdims=True)
            t = jnp.where(d2 == m, iota_kf, float(_K))
            idxf = jnp.min(t, axis=1, keepdims=True)
            idx_cols[c].append(idxf[:, 0].astype(jnp.int32))
            onehot = (iota_kf == idxf).astype(jnp.bfloat16)

            def pick(part, onehot=onehot):
                return jax.lax.dot_general(
                    onehot, part, (((1,), (0,)), ((), ())),
                    preferred_element_type=jnp.float32)

            q = (pick(cb_bf) + pick(cb_mid)) + pick(cb_lo)
            diff = q - r
            sses[c] = sses[c] + jnp.sum(diff * diff, keepdims=True)
            q_st = r + diff
            zqs[c] = zqs[c] + q_st
            rs[c] = r - q_st
    for c in range(n_chunks):
        zq_ref[c * _CH:(c + 1) * _CH, :] = zqs[c]
        idx_ref[c * _CH:(c + 1) * _CH, :] = jnp.stack(idx_cols[c], axis=1)
    sse = sum(sses[1:], sses[0])
    sse_ref[...] = jnp.broadcast_to(sse.reshape(1, 1, 1), (1, 1, 128))


def kernel(z, codebook_0, codebook_1, codebook_2, codebook_3):
    orig_shape = z.shape
    n = orig_shape[0] * orig_shape[1]
    z_flat = z.reshape(n, _D)
    cbs = jnp.stack([codebook_0, codebook_1, codebook_2, codebook_3], axis=0)

    nblk = n // _NB
    zq_flat, idx_flat, sse_parts = pl.pallas_call(
        _rvq_body,
        grid=(nblk,),
        in_specs=[
            pl.BlockSpec((_NB, _D), lambda i: (i, 0)),
            pl.BlockSpec((_NUM_LEVELS, _K, _D), lambda i: (0, 0, 0)),
        ],
        out_specs=[
            pl.BlockSpec((_NB, _D), lambda i: (i, 0)),
            pl.BlockSpec((_NB, _NUM_LEVELS), lambda i: (i, 0)),
            pl.BlockSpec((1, 1, 128), lambda i: (i, 0, 0)),
        ],
        out_shape=[
            jax.ShapeDtypeStruct((n, _D), jnp.float32),
            jax.ShapeDtypeStruct((n, _NUM_LEVELS), jnp.int32),
            jax.ShapeDtypeStruct((nblk, 1, 128), jnp.float32),
        ],
        compiler_params=pltpu.CompilerParams(
            dimension_semantics=("parallel",)),
    )(z_flat, cbs)

    codebook_loss = (jnp.sum(sse_parts[:, 0, 0]) / jnp.float32(n * _D)).astype(jnp.float32)
    commitment_loss = jnp.float32(_BETA) * codebook_loss
    z_q = zq_flat.reshape(orig_shape)
    indices_out = idx_flat.reshape(orig_shape[:-1] + (_NUM_LEVELS,))
    return z_q, indices_out, commitment_loss, codebook_loss
